# Initial kernel scaffold; baseline (speedup 1.0000x reference)
import jax
import jax.numpy as jnp
from jax.experimental import pallas as pl


def kernel(x, edge_index, edge_index2):
    raise NotImplementedError("write your pallas kernel here")


# speedup vs baseline: 5.2765x; 5.2765x over previous
import jax
import jax.numpy as jnp
from jax import lax
from jax.experimental import pallas as pl
from jax.experimental.pallas import tpu as pltpu
from jax.experimental.pallas import tpu_sc as plsc

N_NODES = 10000
D_FEAT = 128
N_EDGES = 320000

NUM_CORES = 2
NUM_TILES = 16
NPASS = 4
FCH = D_FEAT // NPASS
EDGES_PER_TILE = N_EDGES // NUM_TILES
CHUNK = 80
CHUNKS_PER_TILE = EDGES_PER_TILE // CHUNK
ROW_START = 624
ROW_LEN = 640
ZROWS = 128


def _body(x0, x1, x2, x3, dsts_hbm, srcs_hbm, out_hbm,
          dst_idx, src_idx, rows0, rows1, zbuf, acc, sem0, sem1):
    c = lax.axis_index("c")
    s = lax.axis_index("s")

    pltpu.sync_copy(dsts_hbm.at[c, s], dst_idx)
    pltpu.sync_copy(srcs_hbm.at[c, s], src_idx)

    zeros16 = jnp.zeros((16,), jnp.float32)
    zch = FCH // 16

    def zfill(i, carry):
        zbuf[i // zch, pl.ds((i % zch) * 16, 16)] = zeros16
        return carry

    lax.fori_loop(0, ZROWS * zch, zfill, 0)

    for f, xf in enumerate((x0, x1, x2, x3)):
        for k in range(ROW_LEN // ZROWS):
            pltpu.sync_copy(zbuf, acc.at[pl.ds(s * ROW_START + k * ZROWS, ZROWS)])
        plsc.subcore_barrier()

        pltpu.make_async_copy(xf.at[src_idx.at[0]], rows0, sem0).start()

        def step(j, carry):
            j0 = 2 * j
            pltpu.make_async_copy(xf.at[src_idx.at[j0 + 1]], rows1, sem1).start()
            pltpu.make_async_copy(xf.at[src_idx.at[j0]], rows0, sem0).wait()
            pltpu.sync_copy(rows0, acc.at[dst_idx.at[j0]], add=True)

            @pl.when(j < CHUNKS_PER_TILE // 2 - 1)
            def _():
                pltpu.make_async_copy(xf.at[src_idx.at[j0 + 2]], rows0, sem0).start()

            pltpu.make_async_copy(xf.at[src_idx.at[j0 + 1]], rows1, sem1).wait()
            pltpu.sync_copy(rows1, acc.at[dst_idx.at[j0 + 1]], add=True)
            return carry

        lax.fori_loop(0, CHUNKS_PER_TILE // 2, step, 0)

        plsc.subcore_barrier()
        pltpu.sync_copy(acc.at[pl.ds(s * ROW_START, ROW_LEN)],
                        out_hbm.at[c * NPASS + f, pl.ds(s * ROW_START, ROW_LEN)])
        plsc.subcore_barrier()


@jax.jit
def kernel(x, edge_index, edge_index2):
    ei1 = edge_index.astype(jnp.int32)
    ei2 = edge_index2.astype(jnp.int32)
    dsts = jnp.stack([ei1[0], ei2[0]]).reshape(
        NUM_CORES, NUM_TILES, CHUNKS_PER_TILE, CHUNK)
    srcs = jnp.stack([ei1[1], ei2[1]]).reshape(
        NUM_CORES, NUM_TILES, CHUNKS_PER_TILE, CHUNK)
    xs = jnp.moveaxis(x.reshape(N_NODES, NPASS, FCH), 1, 0)

    mesh = plsc.VectorSubcoreMesh(core_axis_name="c", subcore_axis_name="s")
    run = pl.kernel(
        _body,
        out_type=jax.ShapeDtypeStruct((NUM_CORES * NPASS, N_NODES, FCH),
                                      jnp.float32),
        mesh=mesh,
        compiler_params=pltpu.CompilerParams(use_tc_tiling_on_sc=False),
        scratch_types=[
            pltpu.VMEM((CHUNKS_PER_TILE, CHUNK), jnp.int32),
            pltpu.VMEM((CHUNKS_PER_TILE, CHUNK), jnp.int32),
            pltpu.VMEM((CHUNK, FCH), jnp.float32),
            pltpu.VMEM((CHUNK, FCH), jnp.float32),
            pltpu.VMEM((ZROWS, FCH), jnp.float32),
            pltpu.VMEM_SHARED((N_NODES, FCH), jnp.float32),
            pltpu.SemaphoreType.DMA,
            pltpu.SemaphoreType.DMA,
        ],
    )
    out = run(xs[0], xs[1], xs[2], xs[3], dsts, srcs)
    return out.transpose(1, 0, 2).reshape(N_NODES, NUM_CORES * D_FEAT)


# speedup vs baseline: 7.9946x; 1.5151x over previous
import jax
import jax.numpy as jnp
from jax import lax
from jax.experimental import pallas as pl
from jax.experimental.pallas import tpu as pltpu
from jax.experimental.pallas import tpu_sc as plsc

N_NODES = 10000
D_FEAT = 128
N_EDGES = 320000

NUM_CORES = 2
NUM_TILES = 16
NPASS = 2
FCH = D_FEAT // NPASS
EDGES_PER_TILE = N_EDGES // NUM_TILES
CHUNK = 80
CHUNKS_PER_TILE = EDGES_PER_TILE // CHUNK
ROW_START = 624
ROW_LEN = 640
ZROWS = 128


def _body(*refs):
    xfs = refs[:NPASS]
    dsts_hbm, srcs_hbm, out_hbm = refs[NPASS:NPASS + 3]
    dst_idx, src_idx, rows0, rows1, zbuf, acc, sem0, sem1 = refs[NPASS + 3:]
    c = lax.axis_index("c")
    s = lax.axis_index("s")

    pltpu.sync_copy(dsts_hbm.at[c, s], dst_idx)
    pltpu.sync_copy(srcs_hbm.at[c, s], src_idx)

    zeros16 = jnp.zeros((16,), jnp.float32)
    zch = FCH // 16

    def zfill(i, carry):
        zbuf[i // zch, pl.ds((i % zch) * 16, 16)] = zeros16
        return carry

    lax.fori_loop(0, ZROWS * zch, zfill, 0)

    for f, xf in enumerate(xfs):
        for k in range(ROW_LEN // ZROWS):
            pltpu.sync_copy(zbuf, acc.at[pl.ds(s * ROW_START + k * ZROWS, ZROWS)])
        plsc.subcore_barrier()

        pltpu.make_async_copy(xf.at[src_idx.at[0]], rows0, sem0).start()

        def step(j, carry):
            j0 = 2 * j
            pltpu.make_async_copy(xf.at[src_idx.at[j0 + 1]], rows1, sem1).start()
            pltpu.make_async_copy(xf.at[src_idx.at[j0]], rows0, sem0).wait()
            pltpu.sync_copy(rows0, acc.at[dst_idx.at[j0]], add=True)

            @pl.when(j < CHUNKS_PER_TILE // 2 - 1)
            def _():
                pltpu.make_async_copy(xf.at[src_idx.at[j0 + 2]], rows0, sem0).start()

            pltpu.make_async_copy(xf.at[src_idx.at[j0 + 1]], rows1, sem1).wait()
            pltpu.sync_copy(rows1, acc.at[dst_idx.at[j0 + 1]], add=True)
            return carry

        lax.fori_loop(0, CHUNKS_PER_TILE // 2, step, 0)

        plsc.subcore_barrier()
        pltpu.sync_copy(acc.at[pl.ds(s * ROW_START, ROW_LEN)],
                        out_hbm.at[c * NPASS + f, pl.ds(s * ROW_START, ROW_LEN)])
        plsc.subcore_barrier()


@jax.jit
def kernel(x, edge_index, edge_index2):
    ei1 = edge_index.astype(jnp.int32)
    ei2 = edge_index2.astype(jnp.int32)
    dsts = jnp.stack([ei1[0], ei2[0]]).reshape(
        NUM_CORES, NUM_TILES, CHUNKS_PER_TILE, CHUNK)
    srcs = jnp.stack([ei1[1], ei2[1]]).reshape(
        NUM_CORES, NUM_TILES, CHUNKS_PER_TILE, CHUNK)
    xs = jnp.moveaxis(x.reshape(N_NODES, NPASS, FCH), 1, 0)

    mesh = plsc.VectorSubcoreMesh(core_axis_name="c", subcore_axis_name="s")
    run = pl.kernel(
        _body,
        out_type=jax.ShapeDtypeStruct((NUM_CORES * NPASS, N_NODES, FCH),
                                      jnp.float32),
        mesh=mesh,
        compiler_params=pltpu.CompilerParams(use_tc_tiling_on_sc=False),
        scratch_types=[
            pltpu.VMEM((CHUNKS_PER_TILE, CHUNK), jnp.int32),
            pltpu.VMEM((CHUNKS_PER_TILE, CHUNK), jnp.int32),
            pltpu.VMEM((CHUNK, FCH), jnp.float32),
            pltpu.VMEM((CHUNK, FCH), jnp.float32),
            pltpu.VMEM((ZROWS, FCH), jnp.float32),
            pltpu.VMEM_SHARED((N_NODES, FCH), jnp.float32),
            pltpu.SemaphoreType.DMA,
            pltpu.SemaphoreType.DMA,
        ],
    )
    out = run(*xs, dsts, srcs)
    return out.transpose(1, 0, 2).reshape(N_NODES, NUM_CORES * D_FEAT)


# speedup vs baseline: 10.1851x; 1.2740x over previous
import jax
import jax.numpy as jnp
from jax import lax
from jax.experimental import pallas as pl
from jax.experimental.pallas import tpu as pltpu
from jax.experimental.pallas import tpu_sc as plsc

N_NODES = 10000
D_FEAT = 128
N_EDGES = 320000

NUM_CORES = 2
NUM_TILES = 16
NPASS = 2
FCH = D_FEAT // NPASS
EDGES_PER_TILE = N_EDGES // NUM_TILES
CHUNK = 80
CHUNKS_PER_TILE = EDGES_PER_TILE // CHUNK
ROW_START = 624
ROW_LEN = 640
ZROWS = 128


NBUF = 5
ROUNDS = CHUNKS_PER_TILE // NBUF


def _body(*refs):
    xfs = refs[:NPASS]
    dsts_hbm, srcs_hbm, out_hbm = refs[NPASS:NPASS + 3]
    rest = refs[NPASS + 3:]
    dst_idx, src_idx = rest[0], rest[1]
    rows = rest[2:2 + NBUF]
    zbuf, acc = rest[2 + NBUF], rest[3 + NBUF]
    gsem = rest[4 + NBUF:4 + 2 * NBUF]
    ssem = rest[4 + 2 * NBUF:4 + 3 * NBUF]
    c = lax.axis_index("c")
    s = lax.axis_index("s")

    pltpu.sync_copy(dsts_hbm.at[c, s], dst_idx)
    pltpu.sync_copy(srcs_hbm.at[c, s], src_idx)

    zeros16 = jnp.zeros((16,), jnp.float32)
    zch = FCH // 16

    def zfill(i, carry):
        zbuf[i // zch, pl.ds((i % zch) * 16, 16)] = zeros16
        return carry

    lax.fori_loop(0, ZROWS * zch, zfill, 0)

    for f, xf in enumerate(xfs):
        for k in range(ROW_LEN // ZROWS):
            pltpu.sync_copy(zbuf, acc.at[pl.ds(s * ROW_START + k * ZROWS, ZROWS)])
        plsc.subcore_barrier()

        for b in range(NBUF):
            pltpu.make_async_copy(xf.at[src_idx.at[b]], rows[b], gsem[b]).start()

        def step(g, carry):
            base = NBUF * g
            for b in range(NBUF):
                pltpu.make_async_copy(xf.at[src_idx.at[base + b]],
                                      rows[b], gsem[b]).wait()
                pltpu.async_copy(rows[b], acc.at[dst_idx.at[base + b]],
                                 ssem[b], add=True)
            for b in range(NBUF):
                @pl.when(g < ROUNDS - 1)
                def _(b=b):
                    pltpu.make_async_copy(rows[b], acc.at[dst_idx.at[base + b]],
                                          ssem[b]).wait()
                    pltpu.make_async_copy(xf.at[src_idx.at[base + NBUF + b]],
                                          rows[b], gsem[b]).start()
            return carry

        lax.fori_loop(0, ROUNDS, step, 0)
        for b in range(NBUF):
            pltpu.make_async_copy(
                rows[b], acc.at[dst_idx.at[NBUF * (ROUNDS - 1) + b]],
                ssem[b]).wait()

        plsc.subcore_barrier()
        pltpu.sync_copy(acc.at[pl.ds(s * ROW_START, ROW_LEN)],
                        out_hbm.at[c * NPASS + f, pl.ds(s * ROW_START, ROW_LEN)])
        plsc.subcore_barrier()


@jax.jit
def kernel(x, edge_index, edge_index2):
    ei1 = edge_index.astype(jnp.int32)
    ei2 = edge_index2.astype(jnp.int32)
    dsts = jnp.stack([ei1[0], ei2[0]]).reshape(
        NUM_CORES, NUM_TILES, CHUNKS_PER_TILE, CHUNK)
    srcs = jnp.stack([ei1[1], ei2[1]]).reshape(
        NUM_CORES, NUM_TILES, CHUNKS_PER_TILE, CHUNK)
    xs = jnp.moveaxis(x.reshape(N_NODES, NPASS, FCH), 1, 0)

    mesh = plsc.VectorSubcoreMesh(core_axis_name="c", subcore_axis_name="s")
    run = pl.kernel(
        _body,
        out_type=jax.ShapeDtypeStruct((NUM_CORES * NPASS, N_NODES, FCH),
                                      jnp.float32),
        mesh=mesh,
        compiler_params=pltpu.CompilerParams(use_tc_tiling_on_sc=False),
        scratch_types=[
            pltpu.VMEM((CHUNKS_PER_TILE, CHUNK), jnp.int32),
            pltpu.VMEM((CHUNKS_PER_TILE, CHUNK), jnp.int32),
            *[pltpu.VMEM((CHUNK, FCH), jnp.float32) for _ in range(NBUF)],
            pltpu.VMEM((ZROWS, FCH), jnp.float32),
            pltpu.VMEM_SHARED((N_NODES, FCH), jnp.float32),
            *[pltpu.SemaphoreType.DMA for _ in range(2 * NBUF)],
        ],
    )
    out = run(*xs, dsts, srcs)
    return out.transpose(1, 0, 2).reshape(N_NODES, NUM_CORES * D_FEAT)


# speedup vs baseline: 12.0505x; 1.1832x over previous
import jax
import jax.numpy as jnp
from jax import lax
from jax.experimental import pallas as pl
from jax.experimental.pallas import tpu as pltpu
from jax.experimental.pallas import tpu_sc as plsc

N_NODES = 10000
D_FEAT = 128
N_EDGES = 320000

NUM_CORES = 2
NUM_TILES = 16
NPASS = 2
FCH = D_FEAT // NPASS
EDGES_PER_TILE = N_EDGES // NUM_TILES
CHUNK = 80
CHUNKS_PER_TILE = EDGES_PER_TILE // CHUNK
ROW_START = 624
ROW_LEN = 640
ZROWS = 128
NBUF = 5
ROUNDS = CHUNKS_PER_TILE // NBUF


def _body(xa_hbm, xb_hbm, e1_hbm, e2_hbm, out_hbm,
          dst_idx, src_idx, rows0, rows1, rows2, rows3, rows4, zbuf, acc,
          *sems):
    rows = (rows0, rows1, rows2, rows3, rows4)
    gsem, ssem = sems[:NBUF], sems[NBUF:]
    c = lax.axis_index("c")
    s = lax.axis_index("s")

    @pl.when(c == 0)
    def _():
        pltpu.sync_copy(e1_hbm.at[0, pl.ds(s * EDGES_PER_TILE, EDGES_PER_TILE)],
                        dst_idx)
        pltpu.sync_copy(e1_hbm.at[1, pl.ds(s * EDGES_PER_TILE, EDGES_PER_TILE)],
                        src_idx)

    @pl.when(c == 1)
    def _():
        pltpu.sync_copy(e2_hbm.at[0, pl.ds(s * EDGES_PER_TILE, EDGES_PER_TILE)],
                        dst_idx)
        pltpu.sync_copy(e2_hbm.at[1, pl.ds(s * EDGES_PER_TILE, EDGES_PER_TILE)],
                        src_idx)

    zeros16 = jnp.zeros((16,), jnp.float32)
    zch = FCH // 16

    def zfill(i, carry):
        zbuf[i // zch, pl.ds((i % zch) * 16, 16)] = zeros16
        return carry

    lax.fori_loop(0, ZROWS * zch, zfill, 0)

    for f, xf in enumerate((xa_hbm, xb_hbm)):
        fcol = f * FCH

        def gather(i, b, xf=xf):
            return pltpu.make_async_copy(
                xf.at[src_idx.at[pl.ds(i * CHUNK, CHUNK)]],
                rows[b], gsem[b])

        def scatter(i, b):
            return pltpu.make_async_copy(
                rows[b], acc.at[dst_idx.at[pl.ds(i * CHUNK, CHUNK)]], ssem[b])

        for k in range(ROW_LEN // ZROWS):
            pltpu.sync_copy(zbuf, acc.at[pl.ds(s * ROW_START + k * ZROWS, ZROWS)])
        plsc.subcore_barrier()

        for b in range(NBUF):
            gather(b, b).start()

        def step(g, carry):
            base = NBUF * g
            for b in range(NBUF):
                gather(base + b, b).wait()
                pltpu.async_copy(rows[b],
                                 acc.at[dst_idx.at[pl.ds((base + b) * CHUNK,
                                                         CHUNK)]],
                                 ssem[b], add=True)
            for b in range(NBUF):
                @pl.when(g < ROUNDS - 1)
                def _(b=b):
                    scatter(base + b, b).wait()
                    gather(base + NBUF + b, b).start()
            return carry

        lax.fori_loop(0, ROUNDS, step, 0)
        for b in range(NBUF):
            scatter(NBUF * (ROUNDS - 1) + b, b).wait()

        plsc.subcore_barrier()
        pltpu.sync_copy(acc.at[pl.ds(s * ROW_START, ROW_LEN)],
                        out_hbm.at[pl.ds(s * ROW_START, ROW_LEN),
                                   pl.ds(c * D_FEAT + fcol, FCH)])
        plsc.subcore_barrier()


@jax.jit
def kernel(x, edge_index, edge_index2):
    e1 = edge_index.astype(jnp.int32)
    e2 = edge_index2.astype(jnp.int32)

    mesh = plsc.VectorSubcoreMesh(core_axis_name="c", subcore_axis_name="s")
    run = pl.kernel(
        _body,
        out_type=jax.ShapeDtypeStruct((N_NODES, NUM_CORES * D_FEAT),
                                      jnp.float32),
        mesh=mesh,
        compiler_params=pltpu.CompilerParams(use_tc_tiling_on_sc=False),
        scratch_types=[
            pltpu.VMEM((EDGES_PER_TILE,), jnp.int32),
            pltpu.VMEM((EDGES_PER_TILE,), jnp.int32),
            *[pltpu.VMEM((CHUNK, FCH), jnp.float32) for _ in range(NBUF)],
            pltpu.VMEM((ZROWS, FCH), jnp.float32),
            pltpu.VMEM_SHARED((N_NODES, FCH), jnp.float32),
            *[pltpu.SemaphoreType.DMA for _ in range(2 * NBUF)],
        ],
    )
    return run(x[:, :FCH], x[:, FCH:], e1, e2)


# speedup vs baseline: 12.0529x; 1.0002x over previous
import jax
import jax.numpy as jnp
from jax import lax
from jax.experimental import pallas as pl
from jax.experimental.pallas import tpu as pltpu
from jax.experimental.pallas import tpu_sc as plsc

N_NODES = 10000
D_FEAT = 128
N_EDGES = 320000

NUM_CORES = 2
NUM_TILES = 16
NPASS = 2
FCH = D_FEAT // NPASS
EDGES_PER_TILE = N_EDGES // NUM_TILES
CHUNK = 80
CHUNKS_PER_TILE = EDGES_PER_TILE // CHUNK
ROW_START = 624
ROW_LEN = 640
ZROWS = 128
NBUF = 5
ROUNDS = CHUNKS_PER_TILE // NBUF


def _body(xa_hbm, xb_hbm, e1_hbm, e2_hbm, out_hbm,
          dst_idx, src_idx, *rest):
    rows = rest[:NBUF]
    zbuf, acc = rest[NBUF], rest[NBUF + 1]
    sems = rest[NBUF + 2:]
    gsem, ssem = sems[:NBUF], sems[NBUF:]
    c = lax.axis_index("c")
    s = lax.axis_index("s")

    @pl.when(c == 0)
    def _():
        pltpu.sync_copy(e1_hbm.at[0, pl.ds(s * EDGES_PER_TILE, EDGES_PER_TILE)],
                        dst_idx)
        pltpu.sync_copy(e1_hbm.at[1, pl.ds(s * EDGES_PER_TILE, EDGES_PER_TILE)],
                        src_idx)

    @pl.when(c == 1)
    def _():
        pltpu.sync_copy(e2_hbm.at[0, pl.ds(s * EDGES_PER_TILE, EDGES_PER_TILE)],
                        dst_idx)
        pltpu.sync_copy(e2_hbm.at[1, pl.ds(s * EDGES_PER_TILE, EDGES_PER_TILE)],
                        src_idx)

    zeros16 = jnp.zeros((16,), jnp.float32)
    zch = FCH // 16

    def zfill(i, carry):
        zbuf[i // zch, pl.ds((i % zch) * 16, 16)] = zeros16
        return carry

    lax.fori_loop(0, ZROWS * zch, zfill, 0)

    for f, xf in enumerate((xa_hbm, xb_hbm)):
        fcol = f * FCH

        def gather(i, b, xf=xf):
            return pltpu.make_async_copy(
                xf.at[src_idx.at[pl.ds(i * CHUNK, CHUNK)]],
                rows[b], gsem[b])

        def scatter(i, b):
            return pltpu.make_async_copy(
                rows[b], acc.at[dst_idx.at[pl.ds(i * CHUNK, CHUNK)]], ssem[b])

        for k in range(ROW_LEN // ZROWS):
            pltpu.sync_copy(zbuf, acc.at[pl.ds(s * ROW_START + k * ZROWS, ZROWS)])
        plsc.subcore_barrier()

        for b in range(NBUF):
            gather(b, b).start()

        def step(g, carry):
            base = NBUF * g
            for b in range(NBUF):
                gather(base + b, b).wait()
                pltpu.async_copy(rows[b],
                                 acc.at[dst_idx.at[pl.ds((base + b) * CHUNK,
                                                         CHUNK)]],
                                 ssem[b], add=True)
            for b in range(NBUF):
                @pl.when(g < ROUNDS - 1)
                def _(b=b):
                    scatter(base + b, b).wait()
                    gather(base + NBUF + b, b).start()
            return carry

        lax.fori_loop(0, ROUNDS, step, 0)
        for b in range(NBUF):
            scatter(NBUF * (ROUNDS - 1) + b, b).wait()

        plsc.subcore_barrier()
        pltpu.sync_copy(acc.at[pl.ds(s * ROW_START, ROW_LEN)],
                        out_hbm.at[pl.ds(s * ROW_START, ROW_LEN),
                                   pl.ds(c * D_FEAT + fcol, FCH)])
        plsc.subcore_barrier()


@jax.jit
def kernel(x, edge_index, edge_index2):
    e1 = edge_index.astype(jnp.int32)
    e2 = edge_index2.astype(jnp.int32)

    mesh = plsc.VectorSubcoreMesh(core_axis_name="c", subcore_axis_name="s")
    run = pl.kernel(
        _body,
        out_type=jax.ShapeDtypeStruct((N_NODES, NUM_CORES * D_FEAT),
                                      jnp.float32),
        mesh=mesh,
        compiler_params=pltpu.CompilerParams(use_tc_tiling_on_sc=False),
        scratch_types=[
            pltpu.VMEM((EDGES_PER_TILE,), jnp.int32),
            pltpu.VMEM((EDGES_PER_TILE,), jnp.int32),
            *[pltpu.VMEM((CHUNK, FCH), jnp.float32) for _ in range(NBUF)],
            pltpu.VMEM((ZROWS, FCH), jnp.float32),
            pltpu.VMEM_SHARED((N_NODES, FCH), jnp.float32),
            *[pltpu.SemaphoreType.DMA for _ in range(2 * NBUF)],
        ],
    )
    return run(x[:, :FCH], x[:, FCH:], e1, e2)


# speedup vs baseline: 12.1910x; 1.0115x over previous
import jax
import jax.numpy as jnp
from jax import lax
from jax.experimental import pallas as pl
from jax.experimental.pallas import tpu as pltpu
from jax.experimental.pallas import tpu_sc as plsc

N_NODES = 10000
D_FEAT = 128
N_EDGES = 320000

NUM_CORES = 2
NUM_TILES = 16
NPASS = 2
FCH = D_FEAT // NPASS
EDGES_PER_TILE = N_EDGES // NUM_TILES
CHUNK = 80
CHUNKS_PER_TILE = EDGES_PER_TILE // CHUNK
ROW_START = 624
ROW_LEN = 640
ZROWS = 128
NBUF = 5
ROUNDS = CHUNKS_PER_TILE // NBUF


def _body(xa_hbm, xb_hbm, e1_hbm, e2_hbm, out_hbm,
          dst_idx, src_idx, *rest):
    rows = rest[:NBUF]
    zbuf, acc = rest[NBUF], rest[NBUF + 1]
    sems = rest[NBUF + 2:]
    gsem, ssem = sems[:NBUF], sems[NBUF:]
    c = lax.axis_index("c")
    s = lax.axis_index("s")

    @pl.when(c == 0)
    def _():
        pltpu.make_async_copy(
            e1_hbm.at[0, pl.ds(s * EDGES_PER_TILE, EDGES_PER_TILE)],
            dst_idx, sems[0]).start()
        pltpu.make_async_copy(
            e1_hbm.at[1, pl.ds(s * EDGES_PER_TILE, EDGES_PER_TILE)],
            src_idx, sems[1]).start()

    @pl.when(c == 1)
    def _():
        pltpu.make_async_copy(
            e2_hbm.at[0, pl.ds(s * EDGES_PER_TILE, EDGES_PER_TILE)],
            dst_idx, sems[0]).start()
        pltpu.make_async_copy(
            e2_hbm.at[1, pl.ds(s * EDGES_PER_TILE, EDGES_PER_TILE)],
            src_idx, sems[1]).start()

    zeros16 = jnp.zeros((16,), jnp.float32)
    zch = FCH // 16

    def zfill(r, carry):
        for k2 in range(zch):
            zbuf[r, pl.ds(k2 * 16, 16)] = zeros16
        return carry

    lax.fori_loop(0, ZROWS, zfill, 0)
    pltpu.make_async_copy(
        e1_hbm.at[0, pl.ds(s * EDGES_PER_TILE, EDGES_PER_TILE)],
        dst_idx, sems[0]).wait()
    pltpu.make_async_copy(
        e1_hbm.at[1, pl.ds(s * EDGES_PER_TILE, EDGES_PER_TILE)],
        src_idx, sems[1]).wait()

    for f, xf in enumerate((xa_hbm, xb_hbm)):
        fcol = f * FCH

        def gather(i, b, xf=xf):
            return pltpu.make_async_copy(
                xf.at[src_idx.at[pl.ds(i * CHUNK, CHUNK)]],
                rows[b], gsem[b])

        def scatter(i, b):
            return pltpu.make_async_copy(
                rows[b], acc.at[dst_idx.at[pl.ds(i * CHUNK, CHUNK)]], ssem[b])

        for k in range(ROW_LEN // ZROWS):
            pltpu.sync_copy(zbuf, acc.at[pl.ds(s * ROW_START + k * ZROWS, ZROWS)])
        plsc.subcore_barrier()

        for b in range(NBUF):
            gather(b, b).start()

        def step(g, carry):
            base = NBUF * g
            for b in range(NBUF):
                gather(base + b, b).wait()
                pltpu.async_copy(rows[b],
                                 acc.at[dst_idx.at[pl.ds((base + b) * CHUNK,
                                                         CHUNK)]],
                                 ssem[b], add=True)
            for b in range(NBUF):
                @pl.when(g < ROUNDS - 1)
                def _(b=b):
                    scatter(base + b, b).wait()
                    gather(base + NBUF + b, b).start()
            return carry

        lax.fori_loop(0, ROUNDS, step, 0)
        for b in range(NBUF):
            scatter(NBUF * (ROUNDS - 1) + b, b).wait()

        plsc.subcore_barrier()
        pltpu.sync_copy(acc.at[pl.ds(s * ROW_START, ROW_LEN)],
                        out_hbm.at[pl.ds(s * ROW_START, ROW_LEN),
                                   pl.ds(c * D_FEAT + fcol, FCH)])
        if f < NPASS - 1:
            plsc.subcore_barrier()


@jax.jit
def kernel(x, edge_index, edge_index2):
    e1 = edge_index.astype(jnp.int32)
    e2 = edge_index2.astype(jnp.int32)

    mesh = plsc.VectorSubcoreMesh(core_axis_name="c", subcore_axis_name="s")
    run = pl.kernel(
        _body,
        out_type=jax.ShapeDtypeStruct((N_NODES, NUM_CORES * D_FEAT),
                                      jnp.float32),
        mesh=mesh,
        compiler_params=pltpu.CompilerParams(use_tc_tiling_on_sc=False),
        scratch_types=[
            pltpu.VMEM((EDGES_PER_TILE,), jnp.int32),
            pltpu.VMEM((EDGES_PER_TILE,), jnp.int32),
            *[pltpu.VMEM((CHUNK, FCH), jnp.float32) for _ in range(NBUF)],
            pltpu.VMEM((ZROWS, FCH), jnp.float32),
            pltpu.VMEM_SHARED((N_NODES, FCH), jnp.float32),
            *[pltpu.SemaphoreType.DMA for _ in range(2 * NBUF)],
        ],
    )
    return run(x[:, :FCH], x[:, FCH:], e1, e2)
